# baseline (speedup 1.0000x reference)
import functools

import jax
import jax.numpy as jnp
from jax import lax
from jax.experimental import pallas as pl
from jax.experimental.pallas import tpu as pltpu
from jax.experimental.pallas import tpu_sc as plsc

N = 10000
E = 320000
D_IN = 128
H = 128
C = 10
G = 100
GP = 128

NC = 2
NS = 16
NW = NC * NS

CHUNK = 128
EPW = E // NW

EC = 256
FULL_CHUNKS = EPW // EC
REM = EPW % EC

EA = 160
FULL_A = EPW // EA
REM_A = EPW % EA

NODE_CHUNKS = N // CHUNK
NODE_REM = N % CHUNK
_SC_ROUNDS = -(-NODE_CHUNKS // NS)
_NW_ROUNDS = -(-NODE_CHUNKS // NW)

_mesh = plsc.VectorSubcoreMesh(core_axis_name="c", subcore_axis_name="s",
                               num_cores=NC, num_subcores=NS)


def _each_node_chunk(tid, ntiles, fn, fn_rem):
    for t in range(-(-NODE_CHUNKS // ntiles)):
        c = tid + t * ntiles

        @pl.when(c < NODE_CHUNKS)
        def _():
            fn(c * CHUNK)

    if NODE_REM:
        @pl.when(tid == ntiles - 1)
        def _():
            fn_rem(NODE_CHUNKS * CHUNK)


_QUADS = FULL_CHUNKS // 4
_EXTRA = FULL_CHUNKS - 4 * _QUADS


@functools.partial(
    pl.kernel,
    out_type=jax.ShapeDtypeStruct((NC, N, H), jnp.float32),
    mesh=_mesh,
    scratch_types=[
        pltpu.VMEM((EC,), jnp.int32),
        pltpu.VMEM((EC,), jnp.int32),
        pltpu.VMEM((EC,), jnp.int32),
        pltpu.VMEM((EC,), jnp.int32),
        pltpu.VMEM((REM,), jnp.int32),
        pltpu.VMEM((EC, H), jnp.float32),
        pltpu.VMEM_SHARED((N, H), jnp.float32),
        pltpu.SemaphoreType.DMA,
        pltpu.SemaphoreType.DMA,
        pltpu.SemaphoreType.DMA,
        pltpu.SemaphoreType.DMA,
        pltpu.SemaphoreType.DMA,
        pltpu.SemaphoreType.DMA,
        pltpu.SemaphoreType.DMA,
        pltpu.SemaphoreType.DMA,
    ],
)
def _sc_degree(col_hbm, zeros_hbm, ones_hbm, degp_hbm,
               col0, col1, col2, col3, col_r, ones_v, deg_sh,
               sC0, sC1, sC2, sC3, sS0, sS1, sS2, sS3):
    cid = lax.axis_index("c")
    sid = lax.axis_index("s")
    wid = cid * NS + sid

    _each_node_chunk(
        sid, NS,
        lambda off: pltpu.sync_copy(zeros_hbm, deg_sh.at[pl.ds(off, CHUNK)]),
        lambda off: pltpu.sync_copy(zeros_hbm.at[pl.ds(0, NODE_REM)],
                                    deg_sh.at[pl.ds(off, NODE_REM)]))
    pltpu.sync_copy(ones_hbm, ones_v)
    plsc.subcore_barrier()

    ebase = wid * EPW

    pltpu.async_copy(col_hbm.at[pl.ds(ebase, EC)], col0, sC0)
    pltpu.async_copy(col_hbm.at[pl.ds(ebase + EC, EC)], col1, sC1)

    def step(g, carry):
        b0 = ebase + 4 * g * EC

        @pl.when(g > 0)
        def _():
            pltpu.make_async_copy(ones_v, deg_sh.at[col2], sS2).wait()
            pltpu.make_async_copy(ones_v, deg_sh.at[col3], sS3).wait()

        pltpu.async_copy(col_hbm.at[pl.ds(b0 + 2 * EC, EC)], col2, sC2)
        pltpu.async_copy(col_hbm.at[pl.ds(b0 + 3 * EC, EC)], col3, sC3)

        pltpu.make_async_copy(col_hbm.at[pl.ds(b0, EC)], col0, sC0).wait()
        pltpu.async_copy(ones_v, deg_sh.at[col0], sS0, add=True)
        pltpu.make_async_copy(col_hbm.at[pl.ds(b0 + EC, EC)], col1, sC1).wait()
        pltpu.async_copy(ones_v, deg_sh.at[col1], sS1, add=True)

        pltpu.make_async_copy(ones_v, deg_sh.at[col0], sS0).wait()
        pltpu.make_async_copy(ones_v, deg_sh.at[col1], sS1).wait()
        pltpu.async_copy(col_hbm.at[pl.ds(b0 + 4 * EC, EC)], col0, sC0)
        pltpu.async_copy(col_hbm.at[pl.ds(b0 + 5 * EC, EC)], col1, sC1)

        pltpu.make_async_copy(col_hbm.at[pl.ds(b0 + 2 * EC, EC)], col2, sC2).wait()
        pltpu.async_copy(ones_v, deg_sh.at[col2], sS2, add=True)
        pltpu.make_async_copy(col_hbm.at[pl.ds(b0 + 3 * EC, EC)], col3, sC3).wait()
        pltpu.async_copy(ones_v, deg_sh.at[col3], sS3, add=True)
        return carry

    lax.fori_loop(0, _QUADS, step, 0)
    pltpu.make_async_copy(ones_v, deg_sh.at[col2], sS2).wait()
    pltpu.make_async_copy(ones_v, deg_sh.at[col3], sS3).wait()
    assert _EXTRA == 3
    pltpu.make_async_copy(col_hbm.at[pl.ds(0, EC)], col0, sC0).wait()
    pltpu.async_copy(ones_v, deg_sh.at[col0], sS0, add=True)
    pltpu.make_async_copy(col_hbm.at[pl.ds(0, EC)], col1, sC1).wait()
    pltpu.async_copy(ones_v, deg_sh.at[col1], sS1, add=True)
    pltpu.sync_copy(col_hbm.at[pl.ds(ebase + (4 * _QUADS + 2) * EC, EC)], col2)
    pltpu.sync_copy(ones_v, deg_sh.at[col2], add=True)
    pltpu.make_async_copy(ones_v, deg_sh.at[col0], sS0).wait()
    pltpu.make_async_copy(ones_v, deg_sh.at[col1], sS1).wait()

    if REM:
        off = ebase + FULL_CHUNKS * EC
        pltpu.sync_copy(col_hbm.at[pl.ds(off, REM)], col_r)
        pltpu.sync_copy(ones_v.at[pl.ds(0, REM)], deg_sh.at[col_r], add=True)
    plsc.subcore_barrier()

    _each_node_chunk(
        sid, NS,
        lambda off: pltpu.sync_copy(deg_sh.at[pl.ds(off, CHUNK)],
                                    degp_hbm.at[cid, pl.ds(off, CHUNK)]),
        lambda off: pltpu.sync_copy(deg_sh.at[pl.ds(off, NODE_REM)],
                                    degp_hbm.at[cid, pl.ds(off, NODE_REM)]))


_PAIRS = FULL_A // 2
assert FULL_A == 2 * _PAIRS


@functools.partial(
    pl.kernel,
    out_type=jax.ShapeDtypeStruct((NC, N, H), jnp.float32),
    mesh=_mesh,
    scratch_types=[
        pltpu.VMEM((EA,), jnp.int32),
        pltpu.VMEM((EA,), jnp.int32),
        pltpu.VMEM((EA,), jnp.int32),
        pltpu.VMEM((EA,), jnp.int32),
        pltpu.VMEM((REM_A,), jnp.int32),
        pltpu.VMEM((REM_A,), jnp.int32),
        pltpu.VMEM((EA, H), jnp.float32),
        pltpu.VMEM((EA, H), jnp.float32),
        pltpu.VMEM_SHARED((N, H), jnp.float32),
        pltpu.SemaphoreType.DMA,
        pltpu.SemaphoreType.DMA,
        pltpu.SemaphoreType.DMA,
        pltpu.SemaphoreType.DMA,
        pltpu.SemaphoreType.DMA,
        pltpu.SemaphoreType.DMA,
        pltpu.SemaphoreType.DMA,
        pltpu.SemaphoreType.DMA,
    ],
)
def _sc_aggregate(y_hbm, row_hbm, col_hbm, zeros_hbm, aggp_hbm,
                  idx0, idx1, col0, col1, idx_r, col_r, rows0, rows1,
                  acc_sh, sI0, sI1, sC0, sC1, sG0, sG1, sS0, sS1):
    cid = lax.axis_index("c")
    sid = lax.axis_index("s")
    wid = cid * NS + sid

    _each_node_chunk(
        sid, NS,
        lambda off: pltpu.sync_copy(zeros_hbm, acc_sh.at[pl.ds(off, CHUNK)]),
        lambda off: pltpu.sync_copy(zeros_hbm.at[pl.ds(0, NODE_REM)],
                                    acc_sh.at[pl.ds(off, NODE_REM)]))
    plsc.subcore_barrier()

    ebase = wid * EPW

    pltpu.async_copy(row_hbm.at[pl.ds(ebase, EA)], idx0, sI0)
    pltpu.async_copy(row_hbm.at[pl.ds(ebase + EA, EA)], idx1, sI1)

    def step(g, carry):
        base0 = ebase + 2 * g * EA
        base1 = base0 + EA

        @pl.when(g > 0)
        def _():
            pltpu.make_async_copy(rows0, acc_sh.at[col0], sS0).wait()
            pltpu.make_async_copy(rows1, acc_sh.at[col1], sS1).wait()

        pltpu.async_copy(col_hbm.at[pl.ds(base0, EA)], col0, sC0)
        pltpu.async_copy(col_hbm.at[pl.ds(base1, EA)], col1, sC1)

        pltpu.make_async_copy(row_hbm.at[pl.ds(base0, EA)], idx0, sI0).wait()
        pltpu.async_copy(y_hbm.at[idx0], rows0, sG0)
        pltpu.make_async_copy(row_hbm.at[pl.ds(base1, EA)], idx1, sI1).wait()
        pltpu.async_copy(y_hbm.at[idx1], rows1, sG1)

        pltpu.make_async_copy(y_hbm.at[idx0], rows0, sG0).wait()
        pltpu.make_async_copy(col_hbm.at[pl.ds(base0, EA)], col0, sC0).wait()
        pltpu.async_copy(rows0, acc_sh.at[col0], sS0, add=True)

        @pl.when(g + 1 < _PAIRS)
        def _():
            pltpu.async_copy(row_hbm.at[pl.ds(base0 + 2 * EA, EA)], idx0, sI0)

        pltpu.make_async_copy(y_hbm.at[idx1], rows1, sG1).wait()
        pltpu.make_async_copy(col_hbm.at[pl.ds(base1, EA)], col1, sC1).wait()
        pltpu.async_copy(rows1, acc_sh.at[col1], sS1, add=True)

        @pl.when(g + 1 < _PAIRS)
        def _():
            pltpu.async_copy(row_hbm.at[pl.ds(base1 + 2 * EA, EA)], idx1, sI1)

        return carry

    lax.fori_loop(0, _PAIRS, step, 0)
    pltpu.make_async_copy(rows0, acc_sh.at[col0], sS0).wait()
    pltpu.make_async_copy(rows1, acc_sh.at[col1], sS1).wait()

    if REM_A:
        off = ebase + FULL_A * EA
        pltpu.sync_copy(row_hbm.at[pl.ds(off, REM_A)], idx_r)
        pltpu.sync_copy(col_hbm.at[pl.ds(off, REM_A)], col_r)
        rrows = rows0.at[pl.ds(0, REM_A)]
        pltpu.async_copy(y_hbm.at[idx_r], rrows, sG0).wait()
        pltpu.sync_copy(rrows, acc_sh.at[col_r], add=True)
    plsc.subcore_barrier()

    _each_node_chunk(
        sid, NS,
        lambda off: pltpu.sync_copy(acc_sh.at[pl.ds(off, CHUNK)],
                                    aggp_hbm.at[cid, pl.ds(off, CHUNK)]),
        lambda off: pltpu.sync_copy(acc_sh.at[pl.ds(off, NODE_REM)],
                                    aggp_hbm.at[cid, pl.ds(off, NODE_REM)]))


@functools.partial(
    pl.kernel,
    out_type=jax.ShapeDtypeStruct((NC, GP, H), jnp.float32),
    mesh=_mesh,
    scratch_types=[
        pltpu.VMEM((CHUNK,), jnp.int32),
        pltpu.VMEM((NODE_REM,), jnp.int32),
        pltpu.VMEM((CHUNK, H), jnp.float32),
        pltpu.VMEM((NODE_REM, H), jnp.float32),
        pltpu.VMEM_SHARED((GP, H), jnp.float32),
    ],
)
def _sc_pool(x_hbm, batch_hbm, zeros_hbm, poolp_hbm,
             bidx_v, bidx_r, rows_v, rows_r, pool_sh):
    cid = lax.axis_index("c")
    sid = lax.axis_index("s")
    wid = cid * NS + sid

    @pl.when(sid == 0)
    def _():
        pltpu.sync_copy(zeros_hbm, pool_sh)
    plsc.subcore_barrier()

    def body(off):
        pltpu.sync_copy(batch_hbm.at[pl.ds(off, CHUNK)], bidx_v)
        pltpu.sync_copy(x_hbm.at[pl.ds(off, CHUNK)], rows_v)
        pltpu.sync_copy(rows_v, pool_sh.at[bidx_v], add=True)

    def body_rem(off):
        pltpu.sync_copy(batch_hbm.at[pl.ds(off, NODE_REM)], bidx_r)
        pltpu.sync_copy(x_hbm.at[pl.ds(off, NODE_REM)], rows_r)
        pltpu.sync_copy(rows_r, pool_sh.at[bidx_r], add=True)

    _each_node_chunk(wid, NW, body, body_rem)
    plsc.subcore_barrier()

    @pl.when(sid == 0)
    def _():
        pltpu.sync_copy(pool_sh, poolp_hbm.at[cid])


BLK = 1000


def _tc_xw_body(x_ref, wi_ref, bi_ref, w0_ref, xw_ref):
    h = jnp.dot(x_ref[...], wi_ref[...], preferred_element_type=jnp.float32)
    h = h + bi_ref[...]
    xw_ref[...] = jnp.dot(h, w0_ref[...], preferred_element_type=jnp.float32)


def _tc_scale_body(xw_ref, degp_ref, y0_ref, dinv_ref):
    deg = degp_ref[0, :, 0:1] + degp_ref[1, :, 0:1] + 1.0
    dinv = lax.rsqrt(deg)
    dinv_ref[...] = dinv
    y0_ref[...] = dinv * xw_ref[...]


def _tc_mid_body(p_ref, y_ref, dinv_ref, b_ref, g_ref, bb_ref, w_ref, yn_ref):
    dinv = dinv_ref[...]
    t = dinv * (p_ref[0] + p_ref[1] + y_ref[...]) + b_ref[...]
    t = jnp.maximum(t, 0.0)
    m = jnp.mean(t, axis=1, keepdims=True)
    v = jnp.mean((t - m) * (t - m), axis=1, keepdims=True)
    t = (t - m) * lax.rsqrt(v + 1e-5) * g_ref[...] + bb_ref[...]
    yn_ref[...] = dinv * jnp.dot(t, w_ref[...], preferred_element_type=jnp.float32)


def _tc_last_body(p_ref, y_ref, dinv_ref, b_ref, x3_ref):
    x3_ref[...] = dinv_ref[...] * (p_ref[0] + p_ref[1] + y_ref[...]) + b_ref[...]


def _tc_head_body(poolp_ref, batch_ref, wc_ref, bc_ref, emb_ref, logp_ref):
    g_iota = lax.broadcasted_iota(jnp.int32, (GP, 1), 0)
    eq = (batch_ref[...] == g_iota).astype(jnp.float32)
    counts = jnp.maximum(jnp.sum(eq, axis=1, keepdims=True), 1.0)
    mean = (poolp_ref[0] + poolp_ref[1]) / counts
    emb = jnp.dot(mean, wc_ref[...], preferred_element_type=jnp.float32)
    emb = emb + bc_ref[...]
    emb_ref[...] = emb
    s = emb - jnp.max(emb, axis=1, keepdims=True)
    logp_ref[...] = s - jnp.log(jnp.sum(jnp.exp(s), axis=1, keepdims=True))


_rows = lambda i: (i, 0)
_bcast = lambda i: (0, 0)
_p3 = lambda i: (0, i, 0)

_nh_spec = pl.BlockSpec((BLK, H), _rows)
_w_spec = pl.BlockSpec((H, H), _bcast)
_b_spec = pl.BlockSpec((1, H), _bcast)
_degp_spec = pl.BlockSpec((NC, BLK, H), _p3)
_p_spec = pl.BlockSpec((NC, BLK, H), _p3)
_dinv_spec = pl.BlockSpec((BLK, 1), _rows)
_nh_out = jax.ShapeDtypeStruct((N, H), jnp.float32)

_tc_xw = pl.pallas_call(
    _tc_xw_body, grid=(N // BLK,),
    in_specs=[_nh_spec, _w_spec, _b_spec, _w_spec],
    out_specs=_nh_spec, out_shape=_nh_out)

_tc_scale = pl.pallas_call(
    _tc_scale_body, grid=(N // BLK,),
    in_specs=[_nh_spec, _degp_spec],
    out_specs=(_nh_spec, _dinv_spec),
    out_shape=(_nh_out, jax.ShapeDtypeStruct((N, 1), jnp.float32)))

_tc_mid = pl.pallas_call(
    _tc_mid_body, grid=(N // BLK,),
    in_specs=[_p_spec, _nh_spec, _dinv_spec, _b_spec, _b_spec, _b_spec, _w_spec],
    out_specs=_nh_spec, out_shape=_nh_out)

_tc_last = pl.pallas_call(
    _tc_last_body, grid=(N // BLK,),
    in_specs=[_p_spec, _nh_spec, _dinv_spec, _b_spec],
    out_specs=_nh_spec, out_shape=_nh_out)

_tc_head = pl.pallas_call(
    _tc_head_body,
    out_shape=(jax.ShapeDtypeStruct((GP, C), jnp.float32),
               jax.ShapeDtypeStruct((GP, C), jnp.float32)))


def kernel(x, edge_index, batch, ptr, centrality, W_init, b_init, W0, b0,
           W1, b1, W2, b2, ln0_g, ln0_b, ln1_g, ln1_b, W_cls, b_cls):
    row = edge_index[0]
    col = edge_index[1]
    zeros128 = jnp.zeros((CHUNK, H), jnp.float32)
    ones128 = jnp.ones((EC, H), jnp.float32)
    r = lambda a: a.reshape(1, -1)

    degp = _sc_degree(col, zeros128, ones128)
    xw0 = _tc_xw(x, W_init, r(b_init), W0)
    y0, dinv = _tc_scale(xw0, degp)
    agg0 = _sc_aggregate(y0, row, col, zeros128)
    y1 = _tc_mid(agg0, y0, dinv, r(b0), r(ln0_g), r(ln0_b), W1)
    agg1 = _sc_aggregate(y1, row, col, zeros128)
    y2 = _tc_mid(agg1, y1, dinv, r(b1), r(ln1_g), r(ln1_b), W2)
    agg2 = _sc_aggregate(y2, row, col, zeros128)
    x3 = _tc_last(agg2, y2, dinv, r(b2))
    poolp = _sc_pool(x3, batch, zeros128)
    emb, logp = _tc_head(poolp, batch.reshape(1, -1), W_cls, r(b_cls))
    return (emb[:G], logp[:G])


# speedup vs baseline: 1.0179x; 1.0179x over previous
import functools

import jax
import jax.numpy as jnp
from jax import lax
from jax.experimental import pallas as pl
from jax.experimental.pallas import tpu as pltpu
from jax.experimental.pallas import tpu_sc as plsc

N = 10000
E = 320000
D_IN = 128
H = 128
C = 10
G = 100
GP = 128

NC = 2
NS = 16
NW = NC * NS

CHUNK = 128
EPW = E // NW

EC = 256
FULL_CHUNKS = EPW // EC
REM = EPW % EC

EA = 160
FULL_A = EPW // EA
REM_A = EPW % EA

NODE_CHUNKS = N // CHUNK
NODE_REM = N % CHUNK
_SC_ROUNDS = -(-NODE_CHUNKS // NS)
_NW_ROUNDS = -(-NODE_CHUNKS // NW)

_mesh = plsc.VectorSubcoreMesh(core_axis_name="c", subcore_axis_name="s",
                               num_cores=NC, num_subcores=NS)


def _each_node_chunk(tid, ntiles, fn, fn_rem):
    for t in range(-(-NODE_CHUNKS // ntiles)):
        c = tid + t * ntiles

        @pl.when(c < NODE_CHUNKS)
        def _():
            fn(c * CHUNK)

    if NODE_REM:
        @pl.when(tid == ntiles - 1)
        def _():
            fn_rem(NODE_CHUNKS * CHUNK)


_QUADS = FULL_CHUNKS // 4
_EXTRA = FULL_CHUNKS - 4 * _QUADS


@functools.partial(
    pl.kernel,
    out_type=jax.ShapeDtypeStruct((NC, N, H), jnp.float32),
    mesh=_mesh,
    scratch_types=[
        pltpu.VMEM((EC,), jnp.int32),
        pltpu.VMEM((EC,), jnp.int32),
        pltpu.VMEM((EC,), jnp.int32),
        pltpu.VMEM((EC,), jnp.int32),
        pltpu.VMEM((REM,), jnp.int32),
        pltpu.VMEM((EC, H), jnp.float32),
        pltpu.VMEM_SHARED((N, H), jnp.float32),
        pltpu.SemaphoreType.DMA,
        pltpu.SemaphoreType.DMA,
        pltpu.SemaphoreType.DMA,
        pltpu.SemaphoreType.DMA,
        pltpu.SemaphoreType.DMA,
        pltpu.SemaphoreType.DMA,
        pltpu.SemaphoreType.DMA,
        pltpu.SemaphoreType.DMA,
    ],
)
def _sc_degree(col_hbm, zeros_hbm, ones_hbm, degp_hbm,
               col0, col1, col2, col3, col_r, ones_v, deg_sh,
               sC0, sC1, sC2, sC3, sS0, sS1, sS2, sS3):
    cid = lax.axis_index("c")
    sid = lax.axis_index("s")
    wid = cid * NS + sid

    _each_node_chunk(
        sid, NS,
        lambda off: pltpu.sync_copy(zeros_hbm, deg_sh.at[pl.ds(off, CHUNK)]),
        lambda off: pltpu.sync_copy(zeros_hbm.at[pl.ds(0, NODE_REM)],
                                    deg_sh.at[pl.ds(off, NODE_REM)]))
    pltpu.sync_copy(ones_hbm, ones_v)
    plsc.subcore_barrier()

    ebase = wid * EPW

    pltpu.async_copy(col_hbm.at[pl.ds(ebase, EC)], col0, sC0)
    pltpu.async_copy(col_hbm.at[pl.ds(ebase + EC, EC)], col1, sC1)

    def step(g, carry):
        b0 = ebase + 4 * g * EC

        @pl.when(g > 0)
        def _():
            pltpu.make_async_copy(ones_v, deg_sh.at[col2], sS2).wait()
            pltpu.make_async_copy(ones_v, deg_sh.at[col3], sS3).wait()

        pltpu.async_copy(col_hbm.at[pl.ds(b0 + 2 * EC, EC)], col2, sC2)
        pltpu.async_copy(col_hbm.at[pl.ds(b0 + 3 * EC, EC)], col3, sC3)

        pltpu.make_async_copy(col_hbm.at[pl.ds(b0, EC)], col0, sC0).wait()
        pltpu.async_copy(ones_v, deg_sh.at[col0], sS0, add=True)
        pltpu.make_async_copy(col_hbm.at[pl.ds(b0 + EC, EC)], col1, sC1).wait()
        pltpu.async_copy(ones_v, deg_sh.at[col1], sS1, add=True)

        pltpu.make_async_copy(ones_v, deg_sh.at[col0], sS0).wait()
        pltpu.make_async_copy(ones_v, deg_sh.at[col1], sS1).wait()
        pltpu.async_copy(col_hbm.at[pl.ds(b0 + 4 * EC, EC)], col0, sC0)
        pltpu.async_copy(col_hbm.at[pl.ds(b0 + 5 * EC, EC)], col1, sC1)

        pltpu.make_async_copy(col_hbm.at[pl.ds(b0 + 2 * EC, EC)], col2, sC2).wait()
        pltpu.async_copy(ones_v, deg_sh.at[col2], sS2, add=True)
        pltpu.make_async_copy(col_hbm.at[pl.ds(b0 + 3 * EC, EC)], col3, sC3).wait()
        pltpu.async_copy(ones_v, deg_sh.at[col3], sS3, add=True)
        return carry

    lax.fori_loop(0, _QUADS, step, 0)
    pltpu.make_async_copy(ones_v, deg_sh.at[col2], sS2).wait()
    pltpu.make_async_copy(ones_v, deg_sh.at[col3], sS3).wait()
    assert _EXTRA == 3
    pltpu.make_async_copy(col_hbm.at[pl.ds(0, EC)], col0, sC0).wait()
    pltpu.async_copy(ones_v, deg_sh.at[col0], sS0, add=True)
    pltpu.make_async_copy(col_hbm.at[pl.ds(0, EC)], col1, sC1).wait()
    pltpu.async_copy(ones_v, deg_sh.at[col1], sS1, add=True)
    pltpu.sync_copy(col_hbm.at[pl.ds(ebase + (4 * _QUADS + 2) * EC, EC)], col2)
    pltpu.sync_copy(ones_v, deg_sh.at[col2], add=True)
    pltpu.make_async_copy(ones_v, deg_sh.at[col0], sS0).wait()
    pltpu.make_async_copy(ones_v, deg_sh.at[col1], sS1).wait()

    if REM:
        off = ebase + FULL_CHUNKS * EC
        pltpu.sync_copy(col_hbm.at[pl.ds(off, REM)], col_r)
        pltpu.sync_copy(ones_v.at[pl.ds(0, REM)], deg_sh.at[col_r], add=True)
    plsc.subcore_barrier()

    _each_node_chunk(
        sid, NS,
        lambda off: pltpu.sync_copy(deg_sh.at[pl.ds(off, CHUNK)],
                                    degp_hbm.at[cid, pl.ds(off, CHUNK)]),
        lambda off: pltpu.sync_copy(deg_sh.at[pl.ds(off, NODE_REM)],
                                    degp_hbm.at[cid, pl.ds(off, NODE_REM)]))


_PAIRS = FULL_A // 2
assert FULL_A == 2 * _PAIRS


@functools.partial(
    pl.kernel,
    out_type=jax.ShapeDtypeStruct((NC, N, H), jnp.float32),
    mesh=_mesh,
    scratch_types=[
        pltpu.VMEM((EA,), jnp.int32),
        pltpu.VMEM((EA,), jnp.int32),
        pltpu.VMEM((EA,), jnp.int32),
        pltpu.VMEM((EA,), jnp.int32),
        pltpu.VMEM((REM_A,), jnp.int32),
        pltpu.VMEM((REM_A,), jnp.int32),
        pltpu.VMEM((EA, H), jnp.float32),
        pltpu.VMEM((EA, H), jnp.float32),
        pltpu.VMEM_SHARED((N, H), jnp.float32),
        pltpu.SemaphoreType.DMA,
        pltpu.SemaphoreType.DMA,
        pltpu.SemaphoreType.DMA,
        pltpu.SemaphoreType.DMA,
        pltpu.SemaphoreType.DMA,
        pltpu.SemaphoreType.DMA,
        pltpu.SemaphoreType.DMA,
        pltpu.SemaphoreType.DMA,
    ],
)
def _sc_aggregate(y_hbm, row_hbm, col_hbm, zeros_hbm, aggp_hbm,
                  idx0, idx1, col0, col1, idx_r, col_r, rows0, rows1,
                  acc_sh, sI0, sI1, sC0, sC1, sG0, sG1, sS0, sS1):
    cid = lax.axis_index("c")
    sid = lax.axis_index("s")
    wid = cid * NS + sid

    _each_node_chunk(
        sid, NS,
        lambda off: pltpu.sync_copy(zeros_hbm, acc_sh.at[pl.ds(off, CHUNK)]),
        lambda off: pltpu.sync_copy(zeros_hbm.at[pl.ds(0, NODE_REM)],
                                    acc_sh.at[pl.ds(off, NODE_REM)]))
    plsc.subcore_barrier()

    ebase = wid * EPW

    pltpu.async_copy(row_hbm.at[pl.ds(ebase, EA)], idx0, sI0)
    pltpu.async_copy(row_hbm.at[pl.ds(ebase + EA, EA)], idx1, sI1)

    def step(g, carry):
        base0 = ebase + 2 * g * EA
        base1 = base0 + EA

        @pl.when(g > 0)
        def _():
            pltpu.make_async_copy(rows0, acc_sh.at[col0], sS0).wait()
            pltpu.make_async_copy(rows1, acc_sh.at[col1], sS1).wait()

        pltpu.async_copy(col_hbm.at[pl.ds(base0, EA)], col0, sC0)
        pltpu.async_copy(col_hbm.at[pl.ds(base1, EA)], col1, sC1)

        pltpu.make_async_copy(row_hbm.at[pl.ds(base0, EA)], idx0, sI0).wait()
        pltpu.async_copy(y_hbm.at[idx0], rows0, sG0)
        pltpu.make_async_copy(row_hbm.at[pl.ds(base1, EA)], idx1, sI1).wait()
        pltpu.async_copy(y_hbm.at[idx1], rows1, sG1)

        pltpu.make_async_copy(y_hbm.at[idx0], rows0, sG0).wait()
        pltpu.make_async_copy(col_hbm.at[pl.ds(base0, EA)], col0, sC0).wait()
        pltpu.async_copy(rows0, acc_sh.at[col0], sS0, add=True)

        @pl.when(g + 1 < _PAIRS)
        def _():
            pltpu.async_copy(row_hbm.at[pl.ds(base0 + 2 * EA, EA)], idx0, sI0)

        pltpu.make_async_copy(y_hbm.at[idx1], rows1, sG1).wait()
        pltpu.make_async_copy(col_hbm.at[pl.ds(base1, EA)], col1, sC1).wait()
        pltpu.async_copy(rows1, acc_sh.at[col1], sS1, add=True)

        @pl.when(g + 1 < _PAIRS)
        def _():
            pltpu.async_copy(row_hbm.at[pl.ds(base1 + 2 * EA, EA)], idx1, sI1)

        return carry

    lax.fori_loop(0, _PAIRS, step, 0)
    pltpu.make_async_copy(rows0, acc_sh.at[col0], sS0).wait()
    pltpu.make_async_copy(rows1, acc_sh.at[col1], sS1).wait()

    if REM_A:
        off = ebase + FULL_A * EA
        pltpu.sync_copy(row_hbm.at[pl.ds(off, REM_A)], idx_r)
        pltpu.sync_copy(col_hbm.at[pl.ds(off, REM_A)], col_r)
        rrows = rows0.at[pl.ds(0, REM_A)]
        pltpu.async_copy(y_hbm.at[idx_r], rrows, sG0).wait()
        pltpu.sync_copy(rrows, acc_sh.at[col_r], add=True)
    plsc.subcore_barrier()

    _each_node_chunk(
        sid, NS,
        lambda off: pltpu.sync_copy(acc_sh.at[pl.ds(off, CHUNK)],
                                    aggp_hbm.at[cid, pl.ds(off, CHUNK)]),
        lambda off: pltpu.sync_copy(acc_sh.at[pl.ds(off, NODE_REM)],
                                    aggp_hbm.at[cid, pl.ds(off, NODE_REM)]))


BLK = 1000


def _tc_xw_body(x_ref, wi_ref, bi_ref, w0_ref, xw_ref):
    h = jnp.dot(x_ref[...], wi_ref[...], preferred_element_type=jnp.float32)
    h = h + bi_ref[...]
    xw_ref[...] = jnp.dot(h, w0_ref[...], preferred_element_type=jnp.float32)


def _tc_scale_body(xw_ref, degp_ref, y0_ref, dinv_ref):
    deg = degp_ref[0, :, 0:1] + degp_ref[1, :, 0:1] + 1.0
    dinv = lax.rsqrt(deg)
    dinv_ref[...] = dinv
    y0_ref[...] = dinv * xw_ref[...]


def _tc_mid_body(p_ref, y_ref, dinv_ref, b_ref, g_ref, bb_ref, w_ref, yn_ref):
    dinv = dinv_ref[...]
    t = dinv * (p_ref[0] + p_ref[1] + y_ref[...]) + b_ref[...]
    t = jnp.maximum(t, 0.0)
    m = jnp.mean(t, axis=1, keepdims=True)
    v = jnp.mean((t - m) * (t - m), axis=1, keepdims=True)
    t = (t - m) * lax.rsqrt(v + 1e-5) * g_ref[...] + bb_ref[...]
    yn_ref[...] = dinv * jnp.dot(t, w_ref[...], preferred_element_type=jnp.float32)


def _tc_head_body(p_ref, y_ref, dinv_ref, b_ref, batch_ref, wc_ref, bc_ref,
                  emb_ref, logp_ref):
    x3 = dinv_ref[...] * (p_ref[0] + p_ref[1] + y_ref[...]) + b_ref[...]
    g_iota = lax.broadcasted_iota(jnp.int32, (GP, 1), 0)
    eq = (batch_ref[...] == g_iota).astype(jnp.float32)
    counts = jnp.maximum(jnp.sum(eq, axis=1, keepdims=True), 1.0)
    sums = jnp.dot(eq, x3, preferred_element_type=jnp.float32,
                   precision=lax.Precision.HIGHEST)
    mean = sums / counts
    emb = jnp.dot(mean, wc_ref[...], preferred_element_type=jnp.float32)
    emb = emb + bc_ref[...]
    emb_ref[...] = emb
    s = emb - jnp.max(emb, axis=1, keepdims=True)
    logp_ref[...] = s - jnp.log(jnp.sum(jnp.exp(s), axis=1, keepdims=True))


_rows = lambda i: (i, 0)
_bcast = lambda i: (0, 0)
_p3 = lambda i: (0, i, 0)

_nh_spec = pl.BlockSpec((BLK, H), _rows)
_w_spec = pl.BlockSpec((H, H), _bcast)
_b_spec = pl.BlockSpec((1, H), _bcast)
_degp_spec = pl.BlockSpec((NC, BLK, H), _p3)
_p_spec = pl.BlockSpec((NC, BLK, H), _p3)
_dinv_spec = pl.BlockSpec((BLK, 1), _rows)
_nh_out = jax.ShapeDtypeStruct((N, H), jnp.float32)

_tc_xw = pl.pallas_call(
    _tc_xw_body, grid=(N // BLK,),
    in_specs=[_nh_spec, _w_spec, _b_spec, _w_spec],
    out_specs=_nh_spec, out_shape=_nh_out)

_tc_scale = pl.pallas_call(
    _tc_scale_body, grid=(N // BLK,),
    in_specs=[_nh_spec, _degp_spec],
    out_specs=(_nh_spec, _dinv_spec),
    out_shape=(_nh_out, jax.ShapeDtypeStruct((N, 1), jnp.float32)))

_tc_mid = pl.pallas_call(
    _tc_mid_body, grid=(N // BLK,),
    in_specs=[_p_spec, _nh_spec, _dinv_spec, _b_spec, _b_spec, _b_spec, _w_spec],
    out_specs=_nh_spec, out_shape=_nh_out)

_tc_head = pl.pallas_call(
    _tc_head_body,
    out_shape=(jax.ShapeDtypeStruct((GP, C), jnp.float32),
               jax.ShapeDtypeStruct((GP, C), jnp.float32)))


def kernel(x, edge_index, batch, ptr, centrality, W_init, b_init, W0, b0,
           W1, b1, W2, b2, ln0_g, ln0_b, ln1_g, ln1_b, W_cls, b_cls):
    row = edge_index[0]
    col = edge_index[1]
    zeros128 = jnp.zeros((CHUNK, H), jnp.float32)
    ones128 = jnp.ones((EC, H), jnp.float32)
    r = lambda a: a.reshape(1, -1)

    degp = _sc_degree(col, zeros128, ones128)
    xw0 = _tc_xw(x, W_init, r(b_init), W0)
    y0, dinv = _tc_scale(xw0, degp)
    agg0 = _sc_aggregate(y0, row, col, zeros128)
    y1 = _tc_mid(agg0, y0, dinv, r(b0), r(ln0_g), r(ln0_b), W1)
    agg1 = _sc_aggregate(y1, row, col, zeros128)
    y2 = _tc_mid(agg1, y1, dinv, r(b1), r(ln1_g), r(ln1_b), W2)
    agg2 = _sc_aggregate(y2, row, col, zeros128)
    emb, logp = _tc_head(agg2, y2, dinv, r(b2), batch.reshape(1, -1),
                         W_cls, r(b_cls))
    return (emb[:G], logp[:G])


# speedup vs baseline: 1.0314x; 1.0132x over previous
import functools

import jax
import jax.numpy as jnp
from jax import lax
from jax.experimental import pallas as pl
from jax.experimental.pallas import tpu as pltpu
from jax.experimental.pallas import tpu_sc as plsc

N = 10000
E = 320000
D_IN = 128
H = 128
C = 10
G = 100
GP = 128

NC = 2
NS = 16
NW = NC * NS

CHUNK = 128
EPW = E // NW

EC = 256
FULL_CHUNKS = EPW // EC
REM = EPW % EC

EA = 160
FULL_A = EPW // EA
REM_A = EPW % EA

NODE_CHUNKS = N // CHUNK
NODE_REM = N % CHUNK
_SC_ROUNDS = -(-NODE_CHUNKS // NS)
_NW_ROUNDS = -(-NODE_CHUNKS // NW)

_mesh = plsc.VectorSubcoreMesh(core_axis_name="c", subcore_axis_name="s",
                               num_cores=NC, num_subcores=NS)


def _each_node_chunk(tid, ntiles, fn, fn_rem):
    for t in range(-(-NODE_CHUNKS // ntiles)):
        c = tid + t * ntiles

        @pl.when(c < NODE_CHUNKS)
        def _():
            fn(c * CHUNK)

    if NODE_REM:
        @pl.when(tid == ntiles - 1)
        def _():
            fn_rem(NODE_CHUNKS * CHUNK)


_QUADS = FULL_CHUNKS // 4
_EXTRA = FULL_CHUNKS - 4 * _QUADS


@functools.partial(
    pl.kernel,
    out_type=jax.ShapeDtypeStruct((NC, N, H), jnp.float32),
    mesh=_mesh,
    scratch_types=[
        pltpu.VMEM((EC,), jnp.int32),
        pltpu.VMEM((EC,), jnp.int32),
        pltpu.VMEM((EC,), jnp.int32),
        pltpu.VMEM((EC,), jnp.int32),
        pltpu.VMEM((REM,), jnp.int32),
        pltpu.VMEM((EC, H), jnp.float32),
        pltpu.VMEM_SHARED((N, H), jnp.float32),
        pltpu.SemaphoreType.DMA,
        pltpu.SemaphoreType.DMA,
        pltpu.SemaphoreType.DMA,
        pltpu.SemaphoreType.DMA,
        pltpu.SemaphoreType.DMA,
        pltpu.SemaphoreType.DMA,
        pltpu.SemaphoreType.DMA,
        pltpu.SemaphoreType.DMA,
    ],
)
def _sc_degree(col_hbm, zeros_hbm, ones_hbm, degp_hbm,
               col0, col1, col2, col3, col_r, ones_v, deg_sh,
               sC0, sC1, sC2, sC3, sS0, sS1, sS2, sS3):
    cid = lax.axis_index("c")
    sid = lax.axis_index("s")
    wid = cid * NS + sid

    ebase = wid * EPW

    pltpu.async_copy(col_hbm.at[pl.ds(ebase, EC)], col0, sC0)
    pltpu.async_copy(col_hbm.at[pl.ds(ebase + EC, EC)], col1, sC1)

    _each_node_chunk(
        sid, NS,
        lambda off: pltpu.sync_copy(zeros_hbm, deg_sh.at[pl.ds(off, CHUNK)]),
        lambda off: pltpu.sync_copy(zeros_hbm.at[pl.ds(0, NODE_REM)],
                                    deg_sh.at[pl.ds(off, NODE_REM)]))
    pltpu.sync_copy(ones_hbm, ones_v)
    plsc.subcore_barrier()

    def step(g, carry):
        b0 = ebase + 4 * g * EC

        @pl.when(g > 0)
        def _():
            pltpu.make_async_copy(ones_v, deg_sh.at[col2], sS2).wait()
            pltpu.make_async_copy(ones_v, deg_sh.at[col3], sS3).wait()

        pltpu.async_copy(col_hbm.at[pl.ds(b0 + 2 * EC, EC)], col2, sC2)
        pltpu.async_copy(col_hbm.at[pl.ds(b0 + 3 * EC, EC)], col3, sC3)

        pltpu.make_async_copy(col_hbm.at[pl.ds(b0, EC)], col0, sC0).wait()
        pltpu.async_copy(ones_v, deg_sh.at[col0], sS0, add=True)
        pltpu.make_async_copy(col_hbm.at[pl.ds(b0 + EC, EC)], col1, sC1).wait()
        pltpu.async_copy(ones_v, deg_sh.at[col1], sS1, add=True)

        pltpu.make_async_copy(ones_v, deg_sh.at[col0], sS0).wait()
        pltpu.make_async_copy(ones_v, deg_sh.at[col1], sS1).wait()
        pltpu.async_copy(col_hbm.at[pl.ds(b0 + 4 * EC, EC)], col0, sC0)
        pltpu.async_copy(col_hbm.at[pl.ds(b0 + 5 * EC, EC)], col1, sC1)

        pltpu.make_async_copy(col_hbm.at[pl.ds(b0 + 2 * EC, EC)], col2, sC2).wait()
        pltpu.async_copy(ones_v, deg_sh.at[col2], sS2, add=True)
        pltpu.make_async_copy(col_hbm.at[pl.ds(b0 + 3 * EC, EC)], col3, sC3).wait()
        pltpu.async_copy(ones_v, deg_sh.at[col3], sS3, add=True)
        return carry

    lax.fori_loop(0, _QUADS, step, 0)
    pltpu.make_async_copy(ones_v, deg_sh.at[col2], sS2).wait()
    pltpu.make_async_copy(ones_v, deg_sh.at[col3], sS3).wait()
    assert _EXTRA == 3
    pltpu.make_async_copy(col_hbm.at[pl.ds(0, EC)], col0, sC0).wait()
    pltpu.async_copy(ones_v, deg_sh.at[col0], sS0, add=True)
    pltpu.make_async_copy(col_hbm.at[pl.ds(0, EC)], col1, sC1).wait()
    pltpu.async_copy(ones_v, deg_sh.at[col1], sS1, add=True)
    pltpu.sync_copy(col_hbm.at[pl.ds(ebase + (4 * _QUADS + 2) * EC, EC)], col2)
    pltpu.sync_copy(ones_v, deg_sh.at[col2], add=True)
    pltpu.make_async_copy(ones_v, deg_sh.at[col0], sS0).wait()
    pltpu.make_async_copy(ones_v, deg_sh.at[col1], sS1).wait()

    if REM:
        off = ebase + FULL_CHUNKS * EC
        pltpu.sync_copy(col_hbm.at[pl.ds(off, REM)], col_r)
        pltpu.sync_copy(ones_v.at[pl.ds(0, REM)], deg_sh.at[col_r], add=True)
    plsc.subcore_barrier()

    _each_node_chunk(
        sid, NS,
        lambda off: pltpu.sync_copy(deg_sh.at[pl.ds(off, CHUNK)],
                                    degp_hbm.at[cid, pl.ds(off, CHUNK)]),
        lambda off: pltpu.sync_copy(deg_sh.at[pl.ds(off, NODE_REM)],
                                    degp_hbm.at[cid, pl.ds(off, NODE_REM)]))


_PAIRS = FULL_A // 2
assert FULL_A == 2 * _PAIRS


@functools.partial(
    pl.kernel,
    out_type=jax.ShapeDtypeStruct((NC, N, H), jnp.float32),
    mesh=_mesh,
    scratch_types=[
        pltpu.VMEM((EA,), jnp.int32),
        pltpu.VMEM((EA,), jnp.int32),
        pltpu.VMEM((EA,), jnp.int32),
        pltpu.VMEM((EA,), jnp.int32),
        pltpu.VMEM((REM_A,), jnp.int32),
        pltpu.VMEM((REM_A,), jnp.int32),
        pltpu.VMEM((EA, H), jnp.float32),
        pltpu.VMEM((EA, H), jnp.float32),
        pltpu.VMEM_SHARED((N, H), jnp.float32),
        pltpu.SemaphoreType.DMA,
        pltpu.SemaphoreType.DMA,
        pltpu.SemaphoreType.DMA,
        pltpu.SemaphoreType.DMA,
        pltpu.SemaphoreType.DMA,
        pltpu.SemaphoreType.DMA,
        pltpu.SemaphoreType.DMA,
        pltpu.SemaphoreType.DMA,
    ],
)
def _sc_aggregate(y_hbm, row_hbm, col_hbm, zeros_hbm, aggp_hbm,
                  idx0, idx1, col0, col1, idx_r, col_r, rows0, rows1,
                  acc_sh, sI0, sI1, sC0, sC1, sG0, sG1, sS0, sS1):
    cid = lax.axis_index("c")
    sid = lax.axis_index("s")
    wid = cid * NS + sid

    ebase = wid * EPW

    pltpu.async_copy(row_hbm.at[pl.ds(ebase, EA)], idx0, sI0)
    pltpu.async_copy(row_hbm.at[pl.ds(ebase + EA, EA)], idx1, sI1)

    _each_node_chunk(
        sid, NS,
        lambda off: pltpu.sync_copy(zeros_hbm, acc_sh.at[pl.ds(off, CHUNK)]),
        lambda off: pltpu.sync_copy(zeros_hbm.at[pl.ds(0, NODE_REM)],
                                    acc_sh.at[pl.ds(off, NODE_REM)]))

    pltpu.make_async_copy(row_hbm.at[pl.ds(ebase, EA)], idx0, sI0).wait()
    pltpu.async_copy(y_hbm.at[idx0], rows0, sG0)
    pltpu.make_async_copy(row_hbm.at[pl.ds(ebase + EA, EA)], idx1, sI1).wait()
    pltpu.async_copy(y_hbm.at[idx1], rows1, sG1)
    plsc.subcore_barrier()

    def step(g, carry):
        base0 = ebase + 2 * g * EA
        base1 = base0 + EA

        @pl.when(g > 0)
        def _():
            pltpu.make_async_copy(rows0, acc_sh.at[col0], sS0).wait()
            pltpu.make_async_copy(rows1, acc_sh.at[col1], sS1).wait()

        pltpu.async_copy(col_hbm.at[pl.ds(base0, EA)], col0, sC0)
        pltpu.async_copy(col_hbm.at[pl.ds(base1, EA)], col1, sC1)

        @pl.when(g > 0)
        def _():
            pltpu.make_async_copy(row_hbm.at[pl.ds(base0, EA)], idx0, sI0).wait()
            pltpu.async_copy(y_hbm.at[idx0], rows0, sG0)
            pltpu.make_async_copy(row_hbm.at[pl.ds(base1, EA)], idx1, sI1).wait()
            pltpu.async_copy(y_hbm.at[idx1], rows1, sG1)

        pltpu.make_async_copy(y_hbm.at[idx0], rows0, sG0).wait()
        pltpu.make_async_copy(col_hbm.at[pl.ds(base0, EA)], col0, sC0).wait()
        pltpu.async_copy(rows0, acc_sh.at[col0], sS0, add=True)

        @pl.when(g + 1 < _PAIRS)
        def _():
            pltpu.async_copy(row_hbm.at[pl.ds(base0 + 2 * EA, EA)], idx0, sI0)

        pltpu.make_async_copy(y_hbm.at[idx1], rows1, sG1).wait()
        pltpu.make_async_copy(col_hbm.at[pl.ds(base1, EA)], col1, sC1).wait()
        pltpu.async_copy(rows1, acc_sh.at[col1], sS1, add=True)

        @pl.when(g + 1 < _PAIRS)
        def _():
            pltpu.async_copy(row_hbm.at[pl.ds(base1 + 2 * EA, EA)], idx1, sI1)

        return carry

    lax.fori_loop(0, _PAIRS, step, 0)
    pltpu.make_async_copy(rows0, acc_sh.at[col0], sS0).wait()
    pltpu.make_async_copy(rows1, acc_sh.at[col1], sS1).wait()

    if REM_A:
        off = ebase + FULL_A * EA
        pltpu.sync_copy(row_hbm.at[pl.ds(off, REM_A)], idx_r)
        pltpu.sync_copy(col_hbm.at[pl.ds(off, REM_A)], col_r)
        rrows = rows0.at[pl.ds(0, REM_A)]
        pltpu.async_copy(y_hbm.at[idx_r], rrows, sG0).wait()
        pltpu.sync_copy(rrows, acc_sh.at[col_r], add=True)
    plsc.subcore_barrier()

    _each_node_chunk(
        sid, NS,
        lambda off: pltpu.sync_copy(acc_sh.at[pl.ds(off, CHUNK)],
                                    aggp_hbm.at[cid, pl.ds(off, CHUNK)]),
        lambda off: pltpu.sync_copy(acc_sh.at[pl.ds(off, NODE_REM)],
                                    aggp_hbm.at[cid, pl.ds(off, NODE_REM)]))


BLK = 1000


def _tc_init_body(x_ref, wi_ref, bi_ref, w0_ref, degp_ref, y0_ref, dinv_ref):
    h = jnp.dot(x_ref[...], wi_ref[...], preferred_element_type=jnp.float32)
    h = h + bi_ref[...]
    xw = jnp.dot(h, w0_ref[...], preferred_element_type=jnp.float32)
    deg = degp_ref[0, :, 0:1] + degp_ref[1, :, 0:1] + 1.0
    dinv = lax.rsqrt(deg)
    dinv_ref[...] = dinv
    y0_ref[...] = dinv * xw


def _tc_mid_body(p_ref, y_ref, dinv_ref, b_ref, g_ref, bb_ref, w_ref, yn_ref):
    dinv = dinv_ref[...]
    t = dinv * (p_ref[0] + p_ref[1] + y_ref[...]) + b_ref[...]
    t = jnp.maximum(t, 0.0)
    m = jnp.mean(t, axis=1, keepdims=True)
    v = jnp.mean((t - m) * (t - m), axis=1, keepdims=True)
    t = (t - m) * lax.rsqrt(v + 1e-5) * g_ref[...] + bb_ref[...]
    yn_ref[...] = dinv * jnp.dot(t, w_ref[...], preferred_element_type=jnp.float32)


def _tc_head_body(p_ref, y_ref, dinv_ref, b_ref, batch_ref, wc_ref, bc_ref,
                  emb_ref, logp_ref):
    x3 = dinv_ref[...] * (p_ref[0] + p_ref[1] + y_ref[...]) + b_ref[...]
    g_iota = lax.broadcasted_iota(jnp.int32, (GP, 1), 0)
    eq = (batch_ref[...] == g_iota).astype(jnp.float32)
    counts = jnp.maximum(jnp.sum(eq, axis=1, keepdims=True), 1.0)
    sums = jnp.dot(eq, x3, preferred_element_type=jnp.float32,
                   precision=lax.Precision.HIGHEST)
    mean = sums / counts
    emb = jnp.dot(mean, wc_ref[...], preferred_element_type=jnp.float32)
    emb = emb + bc_ref[...]
    emb_ref[...] = emb
    s = emb - jnp.max(emb, axis=1, keepdims=True)
    logp_ref[...] = s - jnp.log(jnp.sum(jnp.exp(s), axis=1, keepdims=True))


_rows = lambda i: (i, 0)
_bcast = lambda i: (0, 0)
_p3 = lambda i: (0, i, 0)

_nh_spec = pl.BlockSpec((BLK, H), _rows)
_w_spec = pl.BlockSpec((H, H), _bcast)
_b_spec = pl.BlockSpec((1, H), _bcast)
_degp_spec = pl.BlockSpec((NC, BLK, H), _p3)
_p_spec = pl.BlockSpec((NC, BLK, H), _p3)
_dinv_spec = pl.BlockSpec((BLK, 1), _rows)
_nh_out = jax.ShapeDtypeStruct((N, H), jnp.float32)

_tc_init = pl.pallas_call(
    _tc_init_body, grid=(N // BLK,),
    in_specs=[_nh_spec, _w_spec, _b_spec, _w_spec, _degp_spec],
    out_specs=(_nh_spec, _dinv_spec),
    out_shape=(_nh_out, jax.ShapeDtypeStruct((N, 1), jnp.float32)))

_tc_mid = pl.pallas_call(
    _tc_mid_body, grid=(N // BLK,),
    in_specs=[_p_spec, _nh_spec, _dinv_spec, _b_spec, _b_spec, _b_spec, _w_spec],
    out_specs=_nh_spec, out_shape=_nh_out)

_tc_head = pl.pallas_call(
    _tc_head_body,
    out_shape=(jax.ShapeDtypeStruct((GP, C), jnp.float32),
               jax.ShapeDtypeStruct((GP, C), jnp.float32)))


def kernel(x, edge_index, batch, ptr, centrality, W_init, b_init, W0, b0,
           W1, b1, W2, b2, ln0_g, ln0_b, ln1_g, ln1_b, W_cls, b_cls):
    row = edge_index[0]
    col = edge_index[1]
    zeros128 = jnp.zeros((CHUNK, H), jnp.float32)
    ones128 = jnp.ones((EC, H), jnp.float32)
    r = lambda a: a.reshape(1, -1)

    degp = _sc_degree(col, zeros128, ones128)
    y0, dinv = _tc_init(x, W_init, r(b_init), W0, degp)
    agg0 = _sc_aggregate(y0, row, col, zeros128)
    y1 = _tc_mid(agg0, y0, dinv, r(b0), r(ln0_g), r(ln0_b), W1)
    agg1 = _sc_aggregate(y1, row, col, zeros128)
    y2 = _tc_mid(agg1, y1, dinv, r(b1), r(ln1_g), r(ln1_b), W2)
    agg2 = _sc_aggregate(y2, row, col, zeros128)
    emb, logp = _tc_head(agg2, y2, dinv, r(b2), batch.reshape(1, -1),
                         W_cls, r(b_cls))
    return (emb[:G], logp[:G])


# speedup vs baseline: 1.0381x; 1.0065x over previous
import functools

import jax
import jax.numpy as jnp
from jax import lax
from jax.experimental import pallas as pl
from jax.experimental.pallas import tpu as pltpu
from jax.experimental.pallas import tpu_sc as plsc

N = 10000
E = 320000
D_IN = 128
H = 128
C = 10
G = 100
GP = 128

NC = 2
NS = 16
NW = NC * NS

CHUNK = 128
EPW = E // NW

EC = 256
FULL_CHUNKS = EPW // EC
REM = EPW % EC

EA = 192
FULL_A = EPW // EA
REM_A = EPW % EA

NODE_CHUNKS = N // CHUNK
NODE_REM = N % CHUNK
_SC_ROUNDS = -(-NODE_CHUNKS // NS)
_NW_ROUNDS = -(-NODE_CHUNKS // NW)

_mesh = plsc.VectorSubcoreMesh(core_axis_name="c", subcore_axis_name="s",
                               num_cores=NC, num_subcores=NS)


def _each_node_chunk(tid, ntiles, fn, fn_rem):
    for t in range(-(-NODE_CHUNKS // ntiles)):
        c = tid + t * ntiles

        @pl.when(c < NODE_CHUNKS)
        def _():
            fn(c * CHUNK)

    if NODE_REM:
        @pl.when(tid == ntiles - 1)
        def _():
            fn_rem(NODE_CHUNKS * CHUNK)


_QUADS = FULL_CHUNKS // 4
_EXTRA = FULL_CHUNKS - 4 * _QUADS


@functools.partial(
    pl.kernel,
    out_type=jax.ShapeDtypeStruct((NC, N, H), jnp.float32),
    mesh=_mesh,
    scratch_types=[
        pltpu.VMEM((EC,), jnp.int32),
        pltpu.VMEM((EC,), jnp.int32),
        pltpu.VMEM((EC,), jnp.int32),
        pltpu.VMEM((EC,), jnp.int32),
        pltpu.VMEM((REM,), jnp.int32),
        pltpu.VMEM((EC, H), jnp.float32),
        pltpu.VMEM_SHARED((N, H), jnp.float32),
        pltpu.SemaphoreType.DMA,
        pltpu.SemaphoreType.DMA,
        pltpu.SemaphoreType.DMA,
        pltpu.SemaphoreType.DMA,
        pltpu.SemaphoreType.DMA,
        pltpu.SemaphoreType.DMA,
        pltpu.SemaphoreType.DMA,
        pltpu.SemaphoreType.DMA,
    ],
)
def _sc_degree(col_hbm, zeros_hbm, ones_hbm, degp_hbm,
               col0, col1, col2, col3, col_r, ones_v, deg_sh,
               sC0, sC1, sC2, sC3, sS0, sS1, sS2, sS3):
    cid = lax.axis_index("c")
    sid = lax.axis_index("s")
    wid = cid * NS + sid

    ebase = wid * EPW

    pltpu.async_copy(col_hbm.at[pl.ds(ebase, EC)], col0, sC0)
    pltpu.async_copy(col_hbm.at[pl.ds(ebase + EC, EC)], col1, sC1)

    _each_node_chunk(
        sid, NS,
        lambda off: pltpu.sync_copy(zeros_hbm, deg_sh.at[pl.ds(off, CHUNK)]),
        lambda off: pltpu.sync_copy(zeros_hbm.at[pl.ds(0, NODE_REM)],
                                    deg_sh.at[pl.ds(off, NODE_REM)]))
    pltpu.sync_copy(ones_hbm, ones_v)
    plsc.subcore_barrier()

    def step(g, carry):
        b0 = ebase + 4 * g * EC

        @pl.when(g > 0)
        def _():
            pltpu.make_async_copy(ones_v, deg_sh.at[col2], sS2).wait()
            pltpu.make_async_copy(ones_v, deg_sh.at[col3], sS3).wait()

        pltpu.async_copy(col_hbm.at[pl.ds(b0 + 2 * EC, EC)], col2, sC2)
        pltpu.async_copy(col_hbm.at[pl.ds(b0 + 3 * EC, EC)], col3, sC3)

        pltpu.make_async_copy(col_hbm.at[pl.ds(b0, EC)], col0, sC0).wait()
        pltpu.async_copy(ones_v, deg_sh.at[col0], sS0, add=True)
        pltpu.make_async_copy(col_hbm.at[pl.ds(b0 + EC, EC)], col1, sC1).wait()
        pltpu.async_copy(ones_v, deg_sh.at[col1], sS1, add=True)

        pltpu.make_async_copy(ones_v, deg_sh.at[col0], sS0).wait()
        pltpu.make_async_copy(ones_v, deg_sh.at[col1], sS1).wait()
        pltpu.async_copy(col_hbm.at[pl.ds(b0 + 4 * EC, EC)], col0, sC0)
        pltpu.async_copy(col_hbm.at[pl.ds(b0 + 5 * EC, EC)], col1, sC1)

        pltpu.make_async_copy(col_hbm.at[pl.ds(b0 + 2 * EC, EC)], col2, sC2).wait()
        pltpu.async_copy(ones_v, deg_sh.at[col2], sS2, add=True)
        pltpu.make_async_copy(col_hbm.at[pl.ds(b0 + 3 * EC, EC)], col3, sC3).wait()
        pltpu.async_copy(ones_v, deg_sh.at[col3], sS3, add=True)
        return carry

    lax.fori_loop(0, _QUADS, step, 0)
    pltpu.make_async_copy(ones_v, deg_sh.at[col2], sS2).wait()
    pltpu.make_async_copy(ones_v, deg_sh.at[col3], sS3).wait()
    assert _EXTRA == 3
    pltpu.make_async_copy(col_hbm.at[pl.ds(0, EC)], col0, sC0).wait()
    pltpu.async_copy(ones_v, deg_sh.at[col0], sS0, add=True)
    pltpu.make_async_copy(col_hbm.at[pl.ds(0, EC)], col1, sC1).wait()
    pltpu.async_copy(ones_v, deg_sh.at[col1], sS1, add=True)
    pltpu.sync_copy(col_hbm.at[pl.ds(ebase + (4 * _QUADS + 2) * EC, EC)], col2)
    pltpu.sync_copy(ones_v, deg_sh.at[col2], add=True)
    pltpu.make_async_copy(ones_v, deg_sh.at[col0], sS0).wait()
    pltpu.make_async_copy(ones_v, deg_sh.at[col1], sS1).wait()

    if REM:
        off = ebase + FULL_CHUNKS * EC
        pltpu.sync_copy(col_hbm.at[pl.ds(off, REM)], col_r)
        pltpu.sync_copy(ones_v.at[pl.ds(0, REM)], deg_sh.at[col_r], add=True)
    plsc.subcore_barrier()

    _each_node_chunk(
        sid, NS,
        lambda off: pltpu.sync_copy(deg_sh.at[pl.ds(off, CHUNK)],
                                    degp_hbm.at[cid, pl.ds(off, CHUNK)]),
        lambda off: pltpu.sync_copy(deg_sh.at[pl.ds(off, NODE_REM)],
                                    degp_hbm.at[cid, pl.ds(off, NODE_REM)]))


_PAIRS = FULL_A // 2
assert FULL_A == 2 * _PAIRS


@functools.partial(
    pl.kernel,
    out_type=jax.ShapeDtypeStruct((NC, N, H), jnp.float32),
    mesh=_mesh,
    scratch_types=[
        pltpu.VMEM((EA,), jnp.int32),
        pltpu.VMEM((EA,), jnp.int32),
        pltpu.VMEM((EA,), jnp.int32),
        pltpu.VMEM((EA,), jnp.int32),
        pltpu.VMEM((REM_A,), jnp.int32),
        pltpu.VMEM((REM_A,), jnp.int32),
        pltpu.VMEM((EA, H), jnp.float32),
        pltpu.VMEM((EA, H), jnp.float32),
        pltpu.VMEM_SHARED((N, H), jnp.float32),
        pltpu.SemaphoreType.DMA,
        pltpu.SemaphoreType.DMA,
        pltpu.SemaphoreType.DMA,
        pltpu.SemaphoreType.DMA,
        pltpu.SemaphoreType.DMA,
        pltpu.SemaphoreType.DMA,
        pltpu.SemaphoreType.DMA,
        pltpu.SemaphoreType.DMA,
    ],
)
def _sc_aggregate(y_hbm, row_hbm, col_hbm, zeros_hbm, aggp_hbm,
                  idx0, idx1, col0, col1, idx_r, col_r, rows0, rows1,
                  acc_sh, sI0, sI1, sC0, sC1, sG0, sG1, sS0, sS1):
    cid = lax.axis_index("c")
    sid = lax.axis_index("s")
    wid = cid * NS + sid

    ebase = wid * EPW

    pltpu.async_copy(row_hbm.at[pl.ds(ebase, EA)], idx0, sI0)
    pltpu.async_copy(row_hbm.at[pl.ds(ebase + EA, EA)], idx1, sI1)

    _each_node_chunk(
        sid, NS,
        lambda off: pltpu.sync_copy(zeros_hbm, acc_sh.at[pl.ds(off, CHUNK)]),
        lambda off: pltpu.sync_copy(zeros_hbm.at[pl.ds(0, NODE_REM)],
                                    acc_sh.at[pl.ds(off, NODE_REM)]))

    pltpu.make_async_copy(row_hbm.at[pl.ds(ebase, EA)], idx0, sI0).wait()
    pltpu.async_copy(y_hbm.at[idx0], rows0, sG0)
    pltpu.make_async_copy(row_hbm.at[pl.ds(ebase + EA, EA)], idx1, sI1).wait()
    pltpu.async_copy(y_hbm.at[idx1], rows1, sG1)
    plsc.subcore_barrier()

    def step(g, carry):
        base0 = ebase + 2 * g * EA
        base1 = base0 + EA

        @pl.when(g > 0)
        def _():
            pltpu.make_async_copy(rows0, acc_sh.at[col0], sS0).wait()
            pltpu.make_async_copy(rows1, acc_sh.at[col1], sS1).wait()

        pltpu.async_copy(col_hbm.at[pl.ds(base0, EA)], col0, sC0)
        pltpu.async_copy(col_hbm.at[pl.ds(base1, EA)], col1, sC1)

        @pl.when(g > 0)
        def _():
            pltpu.make_async_copy(row_hbm.at[pl.ds(base0, EA)], idx0, sI0).wait()
            pltpu.async_copy(y_hbm.at[idx0], rows0, sG0)
            pltpu.make_async_copy(row_hbm.at[pl.ds(base1, EA)], idx1, sI1).wait()
            pltpu.async_copy(y_hbm.at[idx1], rows1, sG1)

        pltpu.make_async_copy(y_hbm.at[idx0], rows0, sG0).wait()
        pltpu.make_async_copy(col_hbm.at[pl.ds(base0, EA)], col0, sC0).wait()
        pltpu.async_copy(rows0, acc_sh.at[col0], sS0, add=True)

        @pl.when(g + 1 < _PAIRS)
        def _():
            pltpu.async_copy(row_hbm.at[pl.ds(base0 + 2 * EA, EA)], idx0, sI0)

        pltpu.make_async_copy(y_hbm.at[idx1], rows1, sG1).wait()
        pltpu.make_async_copy(col_hbm.at[pl.ds(base1, EA)], col1, sC1).wait()
        pltpu.async_copy(rows1, acc_sh.at[col1], sS1, add=True)

        @pl.when(g + 1 < _PAIRS)
        def _():
            pltpu.async_copy(row_hbm.at[pl.ds(base1 + 2 * EA, EA)], idx1, sI1)

        return carry

    lax.fori_loop(0, _PAIRS, step, 0)
    pltpu.make_async_copy(rows0, acc_sh.at[col0], sS0).wait()
    pltpu.make_async_copy(rows1, acc_sh.at[col1], sS1).wait()

    if REM_A:
        off = ebase + FULL_A * EA
        pltpu.sync_copy(row_hbm.at[pl.ds(off, REM_A)], idx_r)
        pltpu.sync_copy(col_hbm.at[pl.ds(off, REM_A)], col_r)
        rrows = rows0.at[pl.ds(0, REM_A)]
        pltpu.async_copy(y_hbm.at[idx_r], rrows, sG0).wait()
        pltpu.sync_copy(rrows, acc_sh.at[col_r], add=True)
    plsc.subcore_barrier()

    _each_node_chunk(
        sid, NS,
        lambda off: pltpu.sync_copy(acc_sh.at[pl.ds(off, CHUNK)],
                                    aggp_hbm.at[cid, pl.ds(off, CHUNK)]),
        lambda off: pltpu.sync_copy(acc_sh.at[pl.ds(off, NODE_REM)],
                                    aggp_hbm.at[cid, pl.ds(off, NODE_REM)]))


BLK = 1000


def _tc_init_body(x_ref, wi_ref, bi_ref, w0_ref, degp_ref, y0_ref, dinv_ref):
    h = jnp.dot(x_ref[...], wi_ref[...], preferred_element_type=jnp.float32)
    h = h + bi_ref[...]
    xw = jnp.dot(h, w0_ref[...], preferred_element_type=jnp.float32)
    deg = degp_ref[0, :, 0:1] + degp_ref[1, :, 0:1] + 1.0
    dinv = lax.rsqrt(deg)
    dinv_ref[...] = dinv
    y0_ref[...] = dinv * xw


def _tc_mid_body(p_ref, y_ref, dinv_ref, b_ref, g_ref, bb_ref, w_ref, yn_ref):
    dinv = dinv_ref[...]
    t = dinv * (p_ref[0] + p_ref[1] + y_ref[...]) + b_ref[...]
    t = jnp.maximum(t, 0.0)
    m = jnp.mean(t, axis=1, keepdims=True)
    v = jnp.mean((t - m) * (t - m), axis=1, keepdims=True)
    t = (t - m) * lax.rsqrt(v + 1e-5) * g_ref[...] + bb_ref[...]
    yn_ref[...] = dinv * jnp.dot(t, w_ref[...], preferred_element_type=jnp.float32)


def _tc_head_body(p_ref, y_ref, dinv_ref, b_ref, batch_ref, wc_ref, bc_ref,
                  emb_ref, logp_ref):
    x3 = dinv_ref[...] * (p_ref[0] + p_ref[1] + y_ref[...]) + b_ref[...]
    g_iota = lax.broadcasted_iota(jnp.int32, (GP, 1), 0)
    eq = (batch_ref[...] == g_iota).astype(jnp.float32)
    counts = jnp.maximum(jnp.sum(eq, axis=1, keepdims=True), 1.0)
    sums = jnp.dot(eq, x3, preferred_element_type=jnp.float32,
                   precision=lax.Precision.HIGHEST)
    mean = sums / counts
    emb = jnp.dot(mean, wc_ref[...], preferred_element_type=jnp.float32)
    emb = emb + bc_ref[...]
    emb_ref[...] = emb
    s = emb - jnp.max(emb, axis=1, keepdims=True)
    logp_ref[...] = s - jnp.log(jnp.sum(jnp.exp(s), axis=1, keepdims=True))


_rows = lambda i: (i, 0)
_bcast = lambda i: (0, 0)
_p3 = lambda i: (0, i, 0)

_nh_spec = pl.BlockSpec((BLK, H), _rows)
_w_spec = pl.BlockSpec((H, H), _bcast)
_b_spec = pl.BlockSpec((1, H), _bcast)
_degp_spec = pl.BlockSpec((NC, BLK, H), _p3)
_p_spec = pl.BlockSpec((NC, BLK, H), _p3)
_dinv_spec = pl.BlockSpec((BLK, 1), _rows)
_nh_out = jax.ShapeDtypeStruct((N, H), jnp.float32)

_tc_init = pl.pallas_call(
    _tc_init_body, grid=(N // BLK,),
    in_specs=[_nh_spec, _w_spec, _b_spec, _w_spec, _degp_spec],
    out_specs=(_nh_spec, _dinv_spec),
    out_shape=(_nh_out, jax.ShapeDtypeStruct((N, 1), jnp.float32)))

_tc_mid = pl.pallas_call(
    _tc_mid_body, grid=(N // BLK,),
    in_specs=[_p_spec, _nh_spec, _dinv_spec, _b_spec, _b_spec, _b_spec, _w_spec],
    out_specs=_nh_spec, out_shape=_nh_out)

_tc_head = pl.pallas_call(
    _tc_head_body,
    out_shape=(jax.ShapeDtypeStruct((GP, C), jnp.float32),
               jax.ShapeDtypeStruct((GP, C), jnp.float32)))


def kernel(x, edge_index, batch, ptr, centrality, W_init, b_init, W0, b0,
           W1, b1, W2, b2, ln0_g, ln0_b, ln1_g, ln1_b, W_cls, b_cls):
    row = edge_index[0]
    col = edge_index[1]
    zeros128 = jnp.zeros((CHUNK, H), jnp.float32)
    ones128 = jnp.ones((EC, H), jnp.float32)
    r = lambda a: a.reshape(1, -1)

    degp = _sc_degree(col, zeros128, ones128)
    y0, dinv = _tc_init(x, W_init, r(b_init), W0, degp)
    agg0 = _sc_aggregate(y0, row, col, zeros128)
    y1 = _tc_mid(agg0, y0, dinv, r(b0), r(ln0_g), r(ln0_b), W1)
    agg1 = _sc_aggregate(y1, row, col, zeros128)
    y2 = _tc_mid(agg1, y1, dinv, r(b1), r(ln1_g), r(ln1_b), W2)
    agg2 = _sc_aggregate(y2, row, col, zeros128)
    emb, logp = _tc_head(agg2, y2, dinv, r(b2), batch.reshape(1, -1),
                         W_cls, r(b_cls))
    return (emb[:G], logp[:G])


# speedup vs baseline: 1.0398x; 1.0016x over previous
import functools

import jax
import jax.numpy as jnp
from jax import lax
from jax.experimental import pallas as pl
from jax.experimental.pallas import tpu as pltpu
from jax.experimental.pallas import tpu_sc as plsc

N = 10000
E = 320000
D_IN = 128
H = 128
C = 10
G = 100
GP = 128

NC = 2
NS = 16
NW = NC * NS

CHUNK = 128
EPW = E // NW

EC = 256
FULL_CHUNKS = EPW // EC
REM = EPW % EC

EA = 192
FULL_A = EPW // EA
REM_A = EPW % EA

NODE_CHUNKS = N // CHUNK
NODE_REM = N % CHUNK

_mesh = plsc.VectorSubcoreMesh(core_axis_name="c", subcore_axis_name="s",
                               num_cores=NC, num_subcores=NS)


def _each_node_chunk(tid, ntiles, fn, fn_rem):
    for t in range(-(-NODE_CHUNKS // ntiles)):
        c = tid + t * ntiles

        @pl.when(c < NODE_CHUNKS)
        def _():
            fn(c * CHUNK)

    if NODE_REM:
        @pl.when(tid == ntiles - 1)
        def _():
            fn_rem(NODE_CHUNKS * CHUNK)


_QUADS = FULL_CHUNKS // 4
_EXTRA = FULL_CHUNKS - 4 * _QUADS


@functools.partial(
    pl.kernel,
    out_type=jax.ShapeDtypeStruct((NC, N, H), jnp.float32),
    mesh=_mesh,
    scratch_types=[
        pltpu.VMEM((EC,), jnp.int32),
        pltpu.VMEM((EC,), jnp.int32),
        pltpu.VMEM((EC,), jnp.int32),
        pltpu.VMEM((EC,), jnp.int32),
        pltpu.VMEM((REM,), jnp.int32),
        pltpu.VMEM((EC, H), jnp.float32),
        pltpu.VMEM_SHARED((N, H), jnp.float32),
        pltpu.SemaphoreType.DMA,
        pltpu.SemaphoreType.DMA,
        pltpu.SemaphoreType.DMA,
        pltpu.SemaphoreType.DMA,
        pltpu.SemaphoreType.DMA,
        pltpu.SemaphoreType.DMA,
        pltpu.SemaphoreType.DMA,
        pltpu.SemaphoreType.DMA,
    ],
)
def _sc_degree(col_hbm, zeros_hbm, ones_hbm, degp_hbm,
               col0, col1, col2, col3, col_r, ones_v, deg_sh,
               sC0, sC1, sC2, sC3, sS0, sS1, sS2, sS3):
    cid = lax.axis_index("c")
    sid = lax.axis_index("s")
    wid = cid * NS + sid

    ebase = wid * EPW

    pltpu.async_copy(col_hbm.at[pl.ds(ebase, EC)], col0, sC0)
    pltpu.async_copy(col_hbm.at[pl.ds(ebase + EC, EC)], col1, sC1)

    _each_node_chunk(
        sid, NS,
        lambda off: pltpu.sync_copy(zeros_hbm, deg_sh.at[pl.ds(off, CHUNK)]),
        lambda off: pltpu.sync_copy(zeros_hbm.at[pl.ds(0, NODE_REM)],
                                    deg_sh.at[pl.ds(off, NODE_REM)]))
    pltpu.sync_copy(ones_hbm, ones_v)
    plsc.subcore_barrier()

    def step(g, carry):
        b0 = ebase + 4 * g * EC

        @pl.when(g > 0)
        def _():
            pltpu.make_async_copy(ones_v, deg_sh.at[col2], sS2).wait()
            pltpu.make_async_copy(ones_v, deg_sh.at[col3], sS3).wait()

        pltpu.async_copy(col_hbm.at[pl.ds(b0 + 2 * EC, EC)], col2, sC2)
        pltpu.async_copy(col_hbm.at[pl.ds(b0 + 3 * EC, EC)], col3, sC3)

        pltpu.make_async_copy(col_hbm.at[pl.ds(b0, EC)], col0, sC0).wait()
        pltpu.async_copy(ones_v, deg_sh.at[col0], sS0, add=True)
        pltpu.make_async_copy(col_hbm.at[pl.ds(b0 + EC, EC)], col1, sC1).wait()
        pltpu.async_copy(ones_v, deg_sh.at[col1], sS1, add=True)

        pltpu.make_async_copy(ones_v, deg_sh.at[col0], sS0).wait()
        pltpu.make_async_copy(ones_v, deg_sh.at[col1], sS1).wait()
        pltpu.async_copy(col_hbm.at[pl.ds(b0 + 4 * EC, EC)], col0, sC0)
        pltpu.async_copy(col_hbm.at[pl.ds(b0 + 5 * EC, EC)], col1, sC1)

        pltpu.make_async_copy(col_hbm.at[pl.ds(b0 + 2 * EC, EC)], col2, sC2).wait()
        pltpu.async_copy(ones_v, deg_sh.at[col2], sS2, add=True)
        pltpu.make_async_copy(col_hbm.at[pl.ds(b0 + 3 * EC, EC)], col3, sC3).wait()
        pltpu.async_copy(ones_v, deg_sh.at[col3], sS3, add=True)
        return carry

    lax.fori_loop(0, _QUADS, step, 0)
    pltpu.make_async_copy(ones_v, deg_sh.at[col2], sS2).wait()
    pltpu.make_async_copy(ones_v, deg_sh.at[col3], sS3).wait()
    assert _EXTRA == 3
    pltpu.make_async_copy(col_hbm.at[pl.ds(0, EC)], col0, sC0).wait()
    pltpu.async_copy(ones_v, deg_sh.at[col0], sS0, add=True)
    pltpu.make_async_copy(col_hbm.at[pl.ds(0, EC)], col1, sC1).wait()
    pltpu.async_copy(ones_v, deg_sh.at[col1], sS1, add=True)
    pltpu.sync_copy(col_hbm.at[pl.ds(ebase + (4 * _QUADS + 2) * EC, EC)], col2)
    pltpu.sync_copy(ones_v, deg_sh.at[col2], add=True)
    pltpu.make_async_copy(ones_v, deg_sh.at[col0], sS0).wait()
    pltpu.make_async_copy(ones_v, deg_sh.at[col1], sS1).wait()

    if REM:
        off = ebase + FULL_CHUNKS * EC
        pltpu.sync_copy(col_hbm.at[pl.ds(off, REM)], col_r)
        pltpu.sync_copy(ones_v.at[pl.ds(0, REM)], deg_sh.at[col_r], add=True)
    plsc.subcore_barrier()

    _each_node_chunk(
        sid, NS,
        lambda off: pltpu.sync_copy(deg_sh.at[pl.ds(off, CHUNK)],
                                    degp_hbm.at[cid, pl.ds(off, CHUNK)]),
        lambda off: pltpu.sync_copy(deg_sh.at[pl.ds(off, NODE_REM)],
                                    degp_hbm.at[cid, pl.ds(off, NODE_REM)]))


_PAIRS = FULL_A // 2
assert FULL_A == 2 * _PAIRS


@functools.partial(
    pl.kernel,
    out_type=jax.ShapeDtypeStruct((NC, N, H), jnp.float32),
    mesh=_mesh,
    scratch_types=[
        pltpu.VMEM((EA,), jnp.int32),
        pltpu.VMEM((EA,), jnp.int32),
        pltpu.VMEM((EA,), jnp.int32),
        pltpu.VMEM((EA,), jnp.int32),
        pltpu.VMEM((REM_A,), jnp.int32),
        pltpu.VMEM((REM_A,), jnp.int32),
        pltpu.VMEM((EA, H), jnp.float32),
        pltpu.VMEM((EA, H), jnp.float32),
        pltpu.VMEM_SHARED((N, H), jnp.float32),
        pltpu.SemaphoreType.DMA,
        pltpu.SemaphoreType.DMA,
        pltpu.SemaphoreType.DMA,
        pltpu.SemaphoreType.DMA,
        pltpu.SemaphoreType.DMA,
        pltpu.SemaphoreType.DMA,
        pltpu.SemaphoreType.DMA,
        pltpu.SemaphoreType.DMA,
    ],
)
def _sc_aggregate(y_hbm, row_hbm, col_hbm, zeros_hbm, aggp_hbm,
                  idx0, idx1, col0, col1, idx_r, col_r, rows0, rows1,
                  acc_sh, sI0, sI1, sC0, sC1, sG0, sG1, sS0, sS1):
    cid = lax.axis_index("c")
    sid = lax.axis_index("s")
    wid = cid * NS + sid

    ebase = wid * EPW

    pltpu.async_copy(row_hbm.at[pl.ds(ebase, EA)], idx0, sI0)
    pltpu.async_copy(row_hbm.at[pl.ds(ebase + EA, EA)], idx1, sI1)

    _each_node_chunk(
        sid, NS,
        lambda off: pltpu.sync_copy(zeros_hbm, acc_sh.at[pl.ds(off, CHUNK)]),
        lambda off: pltpu.sync_copy(zeros_hbm.at[pl.ds(0, NODE_REM)],
                                    acc_sh.at[pl.ds(off, NODE_REM)]))

    pltpu.make_async_copy(row_hbm.at[pl.ds(ebase, EA)], idx0, sI0).wait()
    pltpu.async_copy(y_hbm.at[idx0], rows0, sG0)
    pltpu.make_async_copy(row_hbm.at[pl.ds(ebase + EA, EA)], idx1, sI1).wait()
    pltpu.async_copy(y_hbm.at[idx1], rows1, sG1)
    plsc.subcore_barrier()

    def step(g, carry):
        base0 = ebase + 2 * g * EA
        base1 = base0 + EA

        @pl.when(g > 0)
        def _():
            pltpu.make_async_copy(rows0, acc_sh.at[col0], sS0).wait()
            pltpu.make_async_copy(rows1, acc_sh.at[col1], sS1).wait()

        pltpu.async_copy(col_hbm.at[pl.ds(base0, EA)], col0, sC0)
        pltpu.async_copy(col_hbm.at[pl.ds(base1, EA)], col1, sC1)

        @pl.when(g > 0)
        def _():
            pltpu.make_async_copy(row_hbm.at[pl.ds(base0, EA)], idx0, sI0).wait()
            pltpu.async_copy(y_hbm.at[idx0], rows0, sG0)
            pltpu.make_async_copy(row_hbm.at[pl.ds(base1, EA)], idx1, sI1).wait()
            pltpu.async_copy(y_hbm.at[idx1], rows1, sG1)

        pltpu.make_async_copy(y_hbm.at[idx0], rows0, sG0).wait()
        pltpu.make_async_copy(col_hbm.at[pl.ds(base0, EA)], col0, sC0).wait()
        pltpu.async_copy(rows0, acc_sh.at[col0], sS0, add=True)

        @pl.when(g + 1 < _PAIRS)
        def _():
            pltpu.async_copy(row_hbm.at[pl.ds(base0 + 2 * EA, EA)], idx0, sI0)

        pltpu.make_async_copy(y_hbm.at[idx1], rows1, sG1).wait()
        pltpu.make_async_copy(col_hbm.at[pl.ds(base1, EA)], col1, sC1).wait()
        pltpu.async_copy(rows1, acc_sh.at[col1], sS1, add=True)

        @pl.when(g + 1 < _PAIRS)
        def _():
            pltpu.async_copy(row_hbm.at[pl.ds(base1 + 2 * EA, EA)], idx1, sI1)

        return carry

    lax.fori_loop(0, _PAIRS, step, 0)
    pltpu.make_async_copy(rows0, acc_sh.at[col0], sS0).wait()
    pltpu.make_async_copy(rows1, acc_sh.at[col1], sS1).wait()

    if REM_A:
        off = ebase + FULL_A * EA
        pltpu.sync_copy(row_hbm.at[pl.ds(off, REM_A)], idx_r)
        pltpu.sync_copy(col_hbm.at[pl.ds(off, REM_A)], col_r)
        rrows = rows0.at[pl.ds(0, REM_A)]
        pltpu.async_copy(y_hbm.at[idx_r], rrows, sG0).wait()
        pltpu.sync_copy(rrows, acc_sh.at[col_r], add=True)
    plsc.subcore_barrier()

    _each_node_chunk(
        sid, NS,
        lambda off: pltpu.sync_copy(acc_sh.at[pl.ds(off, CHUNK)],
                                    aggp_hbm.at[cid, pl.ds(off, CHUNK)]),
        lambda off: pltpu.sync_copy(acc_sh.at[pl.ds(off, NODE_REM)],
                                    aggp_hbm.at[cid, pl.ds(off, NODE_REM)]))


BLK = 1000


def _tc_init_body(x_ref, wi_ref, bi_ref, w0_ref, degp_ref, y0_ref, dinv_ref):
    h = jnp.dot(x_ref[...], wi_ref[...], preferred_element_type=jnp.float32)
    h = h + bi_ref[...]
    xw = jnp.dot(h, w0_ref[...], preferred_element_type=jnp.float32)
    deg = degp_ref[0, :, 0:1] + degp_ref[1, :, 0:1] + 1.0
    dinv = lax.rsqrt(deg)
    dinv_ref[...] = dinv
    y0_ref[...] = dinv * xw


def _tc_mid_body(p_ref, y_ref, dinv_ref, b_ref, g_ref, bb_ref, w_ref, yn_ref):
    dinv = dinv_ref[...]
    t = dinv * (p_ref[0] + p_ref[1] + y_ref[...]) + b_ref[...]
    t = jnp.maximum(t, 0.0)
    m = jnp.mean(t, axis=1, keepdims=True)
    v = jnp.mean((t - m) * (t - m), axis=1, keepdims=True)
    t = (t - m) * lax.rsqrt(v + 1e-5) * g_ref[...] + bb_ref[...]
    yn_ref[...] = dinv * jnp.dot(t, w_ref[...], preferred_element_type=jnp.float32)


def _tc_head_body(p_ref, y_ref, dinv_ref, b_ref, batch_ref, wc_ref, bc_ref,
                  emb_ref, logp_ref):
    x3 = dinv_ref[...] * (p_ref[0] + p_ref[1] + y_ref[...]) + b_ref[...]
    g_iota = lax.broadcasted_iota(jnp.int32, (GP, 1), 0)
    eq = (batch_ref[...] == g_iota).astype(jnp.float32)
    counts = jnp.maximum(jnp.sum(eq, axis=1, keepdims=True), 1.0)
    sums = jnp.dot(eq, x3, preferred_element_type=jnp.float32,
                   precision=lax.Precision.HIGHEST)
    mean = sums / counts
    emb = jnp.dot(mean, wc_ref[...], preferred_element_type=jnp.float32)
    emb = emb + bc_ref[...]
    emb_ref[...] = emb
    s = emb - jnp.max(emb, axis=1, keepdims=True)
    logp_ref[...] = s - jnp.log(jnp.sum(jnp.exp(s), axis=1, keepdims=True))


_rows = lambda i: (i, 0)
_bcast = lambda i: (0, 0)
_p3 = lambda i: (0, i, 0)

_nh_spec = pl.BlockSpec((BLK, H), _rows)
_w_spec = pl.BlockSpec((H, H), _bcast)
_b_spec = pl.BlockSpec((1, H), _bcast)
_degp_spec = pl.BlockSpec((NC, BLK, H), _p3)
_p_spec = pl.BlockSpec((NC, BLK, H), _p3)
_dinv_spec = pl.BlockSpec((BLK, 1), _rows)
_nh_out = jax.ShapeDtypeStruct((N, H), jnp.float32)

_tc_init = pl.pallas_call(
    _tc_init_body, grid=(N // BLK,),
    in_specs=[_nh_spec, _w_spec, _b_spec, _w_spec, _degp_spec],
    out_specs=(_nh_spec, _dinv_spec),
    out_shape=(_nh_out, jax.ShapeDtypeStruct((N, 1), jnp.float32)))

_tc_mid = pl.pallas_call(
    _tc_mid_body, grid=(N // BLK,),
    in_specs=[_p_spec, _nh_spec, _dinv_spec, _b_spec, _b_spec, _b_spec, _w_spec],
    out_specs=_nh_spec, out_shape=_nh_out)

_tc_head = pl.pallas_call(
    _tc_head_body,
    out_shape=(jax.ShapeDtypeStruct((GP, C), jnp.float32),
               jax.ShapeDtypeStruct((GP, C), jnp.float32)))


def kernel(x, edge_index, batch, ptr, centrality, W_init, b_init, W0, b0,
           W1, b1, W2, b2, ln0_g, ln0_b, ln1_g, ln1_b, W_cls, b_cls):
    row = edge_index[0]
    col = edge_index[1]
    zeros128 = jnp.zeros((CHUNK, H), jnp.float32)
    ones128 = jnp.ones((EC, H), jnp.float32)
    r = lambda a: a.reshape(1, -1)

    degp = _sc_degree(col, zeros128, ones128)
    y0, dinv = _tc_init(x, W_init, r(b_init), W0, degp)
    agg0 = _sc_aggregate(y0, row, col, zeros128)
    y1 = _tc_mid(agg0, y0, dinv, r(b0), r(ln0_g), r(ln0_b), W1)
    agg1 = _sc_aggregate(y1, row, col, zeros128)
    y2 = _tc_mid(agg1, y1, dinv, r(b1), r(ln1_g), r(ln1_b), W2)
    agg2 = _sc_aggregate(y2, row, col, zeros128)
    emb, logp = _tc_head(agg2, y2, dinv, r(b2), batch.reshape(1, -1),
                         W_cls, r(b_cls))
    return (emb[:G], logp[:G])
